# baseline (speedup 1.0000x reference)
import jax
import jax.numpy as jnp
import numpy as np
from jax.experimental import pallas as pl

_N = 5000
_NC = 91
_IMG_H, _IMG_W = 800.0, 800.0
_SCORE_THRESH = 0.05
_NMS_THRESH = 0.5
_DETS = 100
_TOPK = 1000
_PAD = 1024
_WX, _WY, _WW, _WH = 10.0, 10.0, 5.0, 5.0
_XFORM_CLIP = float(np.log(1000.0 / 16.0))
_LANES = 128
_ROWS_BLK = 1000
_S_BLK = 128


def _score_kernel(logits_ref, dx_ref, dy_ref, dw_ref, dh_ref,
                  px1_ref, py1_ref, px2_ref, py2_ref, score_ref):
    l = logits_ref[...]
    m = jnp.max(l, axis=1, keepdims=True)
    e = jnp.exp(l - m)
    p = e / jnp.sum(e, axis=1, keepdims=True)

    w = px2_ref[...] - px1_ref[...]
    h = py2_ref[...] - py1_ref[...]
    cx = px1_ref[...] + 0.5 * w
    cy = py1_ref[...] + 0.5 * h
    dx = dx_ref[...] / _WX
    dy = dy_ref[...] / _WY
    dw = jnp.minimum(dw_ref[...] / _WW, _XFORM_CLIP)
    dh = jnp.minimum(dh_ref[...] / _WH, _XFORM_CLIP)
    pcx = dx * w + cx
    pcy = dy * h + cy
    pw = jnp.exp(dw) * w
    ph = jnp.exp(dh) * h
    x1 = jnp.clip(pcx - 0.5 * pw, 0.0, _IMG_W)
    y1 = jnp.clip(pcy - 0.5 * ph, 0.0, _IMG_H)
    x2 = jnp.clip(pcx + 0.5 * pw, 0.0, _IMG_W)
    y2 = jnp.clip(pcy + 0.5 * ph, 0.0, _IMG_H)
    valid = (p > _SCORE_THRESH) & ((x2 - x1) >= 0.01) & ((y2 - y1) >= 0.01)
    score_ref[...] = jnp.where(valid, p, -1.0)


def _cdecode_kernel(px1_ref, py1_ref, px2_ref, py2_ref,
                    dx_ref, dy_ref, dw_ref, dh_ref,
                    x1_ref, y1_ref, x2_ref, y2_ref):
    w = px2_ref[...] - px1_ref[...]
    h = py2_ref[...] - py1_ref[...]
    cx = px1_ref[...] + 0.5 * w
    cy = py1_ref[...] + 0.5 * h
    dx = dx_ref[...] / _WX
    dy = dy_ref[...] / _WY
    dw = jnp.minimum(dw_ref[...] / _WW, _XFORM_CLIP)
    dh = jnp.minimum(dh_ref[...] / _WH, _XFORM_CLIP)
    pcx = dx * w + cx
    pcy = dy * h + cy
    pw = jnp.exp(dw) * w
    ph = jnp.exp(dh) * h
    x1_ref[...] = jnp.clip(pcx - 0.5 * pw, 0.0, _IMG_W)
    y1_ref[...] = jnp.clip(pcy - 0.5 * ph, 0.0, _IMG_H)
    x2_ref[...] = jnp.clip(pcx + 0.5 * pw, 0.0, _IMG_W)
    y2_ref[...] = jnp.clip(pcy + 0.5 * ph, 0.0, _IMG_H)


def _smat_kernel(x1r_ref, y1r_ref, x2r_ref, y2r_ref, labr_ref,
                 x1c_ref, y1c_ref, x2c_ref, y2c_ref, labc_ref,
                 s_ref):
    i = pl.program_id(0)
    x1c = x1c_ref[...]
    y1c = y1c_ref[...]
    x2c = x2c_ref[...]
    y2c = y2c_ref[...]
    scale = jnp.maximum(jnp.maximum(jnp.max(x1c), jnp.max(y1c)),
                        jnp.maximum(jnp.max(x2c), jnp.max(y2c))) + 1.0
    offc = labc_ref[...] * scale
    offr = labr_ref[...] * scale
    ox1c, oy1c, ox2c, oy2c = x1c + offc, y1c + offc, x2c + offc, y2c + offc
    ox1r = x1r_ref[...] + offr
    oy1r = y1r_ref[...] + offr
    ox2r = x2r_ref[...] + offr
    oy2r = y2r_ref[...] + offr
    area_r = (ox2r - ox1r) * (oy2r - oy1r)
    area_c = (ox2c - ox1c) * (oy2c - oy1c)
    ltx = jnp.maximum(ox1r, ox1c)
    lty = jnp.maximum(oy1r, oy1c)
    rbx = jnp.minimum(ox2r, ox2c)
    rby = jnp.minimum(oy2r, oy2c)
    ww = jnp.maximum(rbx - ltx, 0.0)
    hh = jnp.maximum(rby - lty, 0.0)
    inter = ww * hh
    iou = inter / (area_r + area_c - inter + 1e-9)
    row = jax.lax.broadcasted_iota(jnp.int32, (_S_BLK, _PAD), 0) + i * _S_BLK
    col = jax.lax.broadcasted_iota(jnp.int32, (_S_BLK, _PAD), 1)
    s_ref[...] = jnp.where((iou > _NMS_THRESH) & (row < col), 1.0, 0.0)


def _nms_kernel(s_ref, sco_ref, x1_ref, y1_ref, x2_ref, y2_ref, lab_ref,
                osc_ref, ox1_ref, oy1_ref, ox2_ref, oy2_ref, olab_ref):
    s_mat = s_ref[...]
    sco = sco_ref[...]
    validf = jnp.where(sco > 0.0, 1.0, 0.0)

    def fp_cond(st):
        return st[1]

    def fp_body(st):
        k, _ = st
        supp = jax.lax.dot_general(
            k, s_mat, (((1,), (0,)), ((), ())),
            preferred_element_type=jnp.float32)
        newk = jnp.where(supp > 0.5, 0.0, validf)
        changed = jnp.sum(jnp.abs(newk - k)) > 0.0
        return newk, changed

    keep, _ = jax.lax.while_loop(fp_cond, fp_body,
                                 (validf, jnp.bool_(True)))

    sh8 = (_PAD // _LANES, _LANES)
    vals = jnp.reshape(jnp.where(keep > 0.5, sco, -1.0), sh8)
    fidx8 = (jax.lax.broadcasted_iota(jnp.int32, sh8, 0) * _LANES
             + jax.lax.broadcasted_iota(jnp.int32, sh8, 1))
    iota_o = jax.lax.broadcasted_iota(jnp.int32, (1, _LANES), 1)
    x1 = jnp.reshape(x1_ref[...], sh8)
    y1 = jnp.reshape(y1_ref[...], sh8)
    x2 = jnp.reshape(x2_ref[...], sh8)
    y2 = jnp.reshape(y2_ref[...], sh8)
    lab = jnp.reshape(lab_ref[...], sh8)
    zero_o = jnp.zeros((1, _LANES), jnp.float32)

    def sel_body(t, st):
        vals, a_s, a_x1, a_y1, a_x2, a_y2, a_l = st
        m = jnp.max(vals)
        oh = (vals == m) & (fidx8 == jnp.min(
            jnp.where(vals == m, fidx8, _PAD + 1)))
        oho = jnp.where(iota_o == t, 1.0, 0.0)
        a_s = a_s + oho * m
        a_x1 = a_x1 + oho * jnp.sum(jnp.where(oh, x1, 0.0))
        a_y1 = a_y1 + oho * jnp.sum(jnp.where(oh, y1, 0.0))
        a_x2 = a_x2 + oho * jnp.sum(jnp.where(oh, x2, 0.0))
        a_y2 = a_y2 + oho * jnp.sum(jnp.where(oh, y2, 0.0))
        a_l = a_l + oho * jnp.sum(jnp.where(oh, lab, 0.0))
        vals = jnp.where(oh, -jnp.inf, vals)
        return vals, a_s, a_x1, a_y1, a_x2, a_y2, a_l

    st = (vals, zero_o, zero_o, zero_o, zero_o, zero_o, zero_o)
    st = jax.lax.fori_loop(0, _DETS, sel_body, st)
    _, a_s, a_x1, a_y1, a_x2, a_y2, a_l = st
    osc_ref[...] = a_s
    ox1_ref[...] = a_x1
    oy1_ref[...] = a_y1
    ox2_ref[...] = a_x2
    oy2_ref[...] = a_y2
    olab_ref[...] = a_l


def _select_kernel(score_ref, t_ref, c_ref):
    s = score_ref[...]
    lane = jax.lax.broadcasted_iota(jnp.int32, s.shape, 1)
    row = jax.lax.broadcasted_iota(jnp.int32, s.shape, 0)
    lane_ok = (lane >= 1) & (lane < _NC)
    kbits = jax.lax.bitcast_convert_type(s, jnp.int32)
    int_min = jnp.int32(-2147483648)
    keys = jnp.where(lane_ok, kbits, int_min)
    fidx = jnp.where(lane_ok, row * (_NC - 1) + (lane - 1),
                     jnp.int32(2 ** 30))

    def bs1(_, st):
        lo, hi = st
        mid = (lo >> 1) + (hi >> 1) + (lo & hi & 1)
        cnt = jnp.sum((keys > mid).astype(jnp.int32))
        below = cnt < _TOPK
        return jnp.where(below, lo, mid + 1), jnp.where(below, mid, hi)

    lo, hi = jax.lax.fori_loop(
        0, 32, bs1, (int_min, jnp.int32(2147483647)))
    t = lo
    c_gt = jnp.sum((keys > t).astype(jnp.int32))
    m = _TOPK - c_gt
    tie = keys == t

    def bs2(_, st):
        lo, hi = st
        mid = (lo + hi) >> 1
        cnt = jnp.sum((tie & (fidx <= mid)).astype(jnp.int32))
        enough = cnt >= m
        return jnp.where(enough, lo, mid + 1), jnp.where(enough, mid, hi)

    lo2, _ = jax.lax.fori_loop(
        0, 19, bs2, (jnp.int32(0), jnp.int32(_N * (_NC - 1) - 1)))
    t_ref[...] = jnp.reshape(t, (1, 1))
    c_ref[...] = jnp.reshape(lo2, (1, 1))


def kernel(class_logits, box_regression, proposals):
    f32 = jnp.float32
    rel = box_regression.reshape(_N, _NC, 4)
    dx = rel[..., 0]
    dy = rel[..., 1]
    dw = rel[..., 2]
    dh = rel[..., 3]
    px1 = proposals[:, 0:1]
    py1 = proposals[:, 1:2]
    px2 = proposals[:, 2:3]
    py2 = proposals[:, 3:4]

    n_blk = _N // _ROWS_BLK
    row_spec = pl.BlockSpec((_ROWS_BLK, _NC), lambda i: (i, 0))
    col1_spec = pl.BlockSpec((_ROWS_BLK, 1), lambda i: (i, 0))
    score = pl.pallas_call(
        _score_kernel,
        grid=(n_blk,),
        in_specs=[row_spec] * 5 + [col1_spec] * 4,
        out_specs=row_spec,
        out_shape=jax.ShapeDtypeStruct((_N, _NC), f32),
    )(class_logits, dx, dy, dw, dh, px1, py1, px2, py2)

    scores_flat = score[:, 1:_NC].reshape(-1)

    t_arr, c_arr = pl.pallas_call(
        _select_kernel,
        out_shape=[jax.ShapeDtypeStruct((1, 1), jnp.int32)] * 2,
    )(score)
    t_key = t_arr[0, 0]
    c_cut = c_arr[0, 0]
    keys_flat = jax.lax.bitcast_convert_type(scores_flat, jnp.int32)
    pred = (keys_flat > t_key) | (
        (keys_flat == t_key)
        & (jnp.arange(keys_flat.shape[0], dtype=jnp.int32) <= c_cut))
    cidx = jnp.nonzero(pred, size=_TOPK, fill_value=0)[0].astype(jnp.int32)
    top_scores, pos = jax.lax.top_k(scores_flat[cidx], _TOPK)
    top_idx = cidx[pos]
    ridx = top_idx // (_NC - 1)
    cls = top_idx % (_NC - 1) + 1
    tlab = cls.astype(f32)
    brf = box_regression.reshape(-1)
    base = ridx * (_NC * 4) + cls * 4
    prop_c = proposals[ridx]

    npad = _PAD - _TOPK
    def cpad(v, cval=0.0):
        return jnp.pad(v, (0, npad), constant_values=cval).reshape(1, _PAD)
    sco_c = cpad(top_scores, -1.0)
    lab_c = cpad(tlab)
    small = jax.ShapeDtypeStruct((1, _PAD), f32)
    x1c, y1c, x2c, y2c = pl.pallas_call(
        _cdecode_kernel,
        out_shape=[small] * 4,
    )(cpad(prop_c[:, 0]), cpad(prop_c[:, 1]),
      cpad(prop_c[:, 2]), cpad(prop_c[:, 3]),
      cpad(brf[base]), cpad(brf[base + 1]),
      cpad(brf[base + 2]), cpad(brf[base + 3]))
    x1r, y1r = x1c.reshape(_PAD, 1), y1c.reshape(_PAD, 1)
    x2r, y2r = x2c.reshape(_PAD, 1), y2c.reshape(_PAD, 1)
    lab_r = lab_c.reshape(_PAD, 1)

    rblk = pl.BlockSpec((_S_BLK, 1), lambda i: (i, 0))
    cblk = pl.BlockSpec((1, _PAD), lambda i: (0, 0))
    s_mat = pl.pallas_call(
        _smat_kernel,
        grid=(_PAD // _S_BLK,),
        in_specs=[rblk] * 5 + [cblk] * 5,
        out_specs=pl.BlockSpec((_S_BLK, _PAD), lambda i: (i, 0)),
        out_shape=jax.ShapeDtypeStruct((_PAD, _PAD), f32),
    )(x1r, y1r, x2r, y2r, lab_r, x1c, y1c, x2c, y2c, lab_c)

    out_small = jax.ShapeDtypeStruct((1, _LANES), f32)
    osc, ox1, oy1, ox2, oy2, olab = pl.pallas_call(
        _nms_kernel,
        out_shape=[out_small] * 6,
    )(s_mat, sco_c, x1c, y1c, x2c, y2c, lab_c)

    final_scores = osc[0, :_DETS]
    final_boxes = jnp.stack(
        [ox1[0, :_DETS], oy1[0, :_DETS], ox2[0, :_DETS], oy2[0, :_DETS]],
        axis=-1)
    final_labels = olab[0, :_DETS].astype(jnp.int32)
    return final_boxes, final_scores, final_labels


# speedup vs baseline: 1.1036x; 1.1036x over previous
import jax
import jax.numpy as jnp
import numpy as np
from jax.experimental import pallas as pl

_N = 5000
_NC = 91
_IMG_H, _IMG_W = 800.0, 800.0
_SCORE_THRESH = 0.05
_NMS_THRESH = 0.5
_DETS = 100
_TOPK = 1000
_PAD = 1024
_WX, _WY, _WW, _WH = 10.0, 10.0, 5.0, 5.0
_XFORM_CLIP = float(np.log(1000.0 / 16.0))
_LANES = 128
_ROWS_BLK = 1000
_S_BLK = 128


def _decode_kernel(logits_ref, dx_ref, dy_ref, dw_ref, dh_ref,
                   px1_ref, py1_ref, px2_ref, py2_ref,
                   score_ref, x1_ref, y1_ref, x2_ref, y2_ref):
    l = logits_ref[...]
    m = jnp.max(l, axis=1, keepdims=True)
    e = jnp.exp(l - m)
    p = e / jnp.sum(e, axis=1, keepdims=True)

    w = px2_ref[...] - px1_ref[...]
    h = py2_ref[...] - py1_ref[...]
    cx = px1_ref[...] + 0.5 * w
    cy = py1_ref[...] + 0.5 * h
    dx = dx_ref[...] / _WX
    dy = dy_ref[...] / _WY
    dw = jnp.minimum(dw_ref[...] / _WW, _XFORM_CLIP)
    dh = jnp.minimum(dh_ref[...] / _WH, _XFORM_CLIP)
    pcx = dx * w + cx
    pcy = dy * h + cy
    pw = jnp.exp(dw) * w
    ph = jnp.exp(dh) * h
    x1 = jnp.clip(pcx - 0.5 * pw, 0.0, _IMG_W)
    y1 = jnp.clip(pcy - 0.5 * ph, 0.0, _IMG_H)
    x2 = jnp.clip(pcx + 0.5 * pw, 0.0, _IMG_W)
    y2 = jnp.clip(pcy + 0.5 * ph, 0.0, _IMG_H)
    valid = (p > _SCORE_THRESH) & ((x2 - x1) >= 0.01) & ((y2 - y1) >= 0.01)
    score_ref[...] = jnp.where(valid, p, -1.0)
    x1_ref[...] = x1
    y1_ref[...] = y1
    x2_ref[...] = x2
    y2_ref[...] = y2


def _smat_kernel(x1r_ref, y1r_ref, x2r_ref, y2r_ref, labr_ref,
                 x1c_ref, y1c_ref, x2c_ref, y2c_ref, labc_ref,
                 s_ref):
    i = pl.program_id(0)
    x1c = x1c_ref[...]
    y1c = y1c_ref[...]
    x2c = x2c_ref[...]
    y2c = y2c_ref[...]
    scale = jnp.maximum(jnp.maximum(jnp.max(x1c), jnp.max(y1c)),
                        jnp.maximum(jnp.max(x2c), jnp.max(y2c))) + 1.0
    offc = labc_ref[...] * scale
    offr = labr_ref[...] * scale
    ox1c, oy1c, ox2c, oy2c = x1c + offc, y1c + offc, x2c + offc, y2c + offc
    ox1r = x1r_ref[...] + offr
    oy1r = y1r_ref[...] + offr
    ox2r = x2r_ref[...] + offr
    oy2r = y2r_ref[...] + offr
    area_r = (ox2r - ox1r) * (oy2r - oy1r)
    area_c = (ox2c - ox1c) * (oy2c - oy1c)
    ltx = jnp.maximum(ox1r, ox1c)
    lty = jnp.maximum(oy1r, oy1c)
    rbx = jnp.minimum(ox2r, ox2c)
    rby = jnp.minimum(oy2r, oy2c)
    ww = jnp.maximum(rbx - ltx, 0.0)
    hh = jnp.maximum(rby - lty, 0.0)
    inter = ww * hh
    iou = inter / (area_r + area_c - inter + 1e-9)
    row = jax.lax.broadcasted_iota(jnp.int32, (_S_BLK, _PAD), 0) + i * _S_BLK
    col = jax.lax.broadcasted_iota(jnp.int32, (_S_BLK, _PAD), 1)
    s_ref[...] = jnp.where((iou > _NMS_THRESH) & (row < col), 1.0, 0.0)


def _nms_kernel(s_ref, sco_ref, x1_ref, y1_ref, x2_ref, y2_ref, lab_ref,
                osc_ref, ox1_ref, oy1_ref, ox2_ref, oy2_ref, olab_ref):
    s_mat = s_ref[...]
    sco = sco_ref[...]
    validf = jnp.where(sco > 0.0, 1.0, 0.0)

    def fp_cond(st):
        return st[1]

    def fp_body(st):
        k, _ = st
        supp = jax.lax.dot_general(
            k, s_mat, (((1,), (0,)), ((), ())),
            preferred_element_type=jnp.float32)
        newk = jnp.where(supp > 0.5, 0.0, validf)
        changed = jnp.sum(jnp.abs(newk - k)) > 0.0
        return newk, changed

    keep, _ = jax.lax.while_loop(fp_cond, fp_body,
                                 (validf, jnp.bool_(True)))

    vals = jnp.where(keep > 0.5, sco, -1.0)
    iota_l = jax.lax.broadcasted_iota(jnp.int32, (1, _PAD), 1)
    iota_o = jax.lax.broadcasted_iota(jnp.int32, (1, _LANES), 1)
    x1 = x1_ref[...]
    y1 = y1_ref[...]
    x2 = x2_ref[...]
    y2 = y2_ref[...]
    lab = lab_ref[...]
    zero_o = jnp.zeros((1, _LANES), jnp.float32)

    def sel_body(t, st):
        vals, a_s, a_x1, a_y1, a_x2, a_y2, a_l = st
        m = jnp.max(vals, keepdims=True)
        idx = jnp.min(jnp.where(vals == m, iota_l, _PAD + 1), keepdims=True)
        oh = jnp.where(iota_l == idx, 1.0, 0.0)
        oho = jnp.where(iota_o == t, 1.0, 0.0)
        a_s = a_s + oho * m
        a_x1 = a_x1 + oho * jnp.sum(oh * x1, keepdims=True)
        a_y1 = a_y1 + oho * jnp.sum(oh * y1, keepdims=True)
        a_x2 = a_x2 + oho * jnp.sum(oh * x2, keepdims=True)
        a_y2 = a_y2 + oho * jnp.sum(oh * y2, keepdims=True)
        a_l = a_l + oho * jnp.sum(oh * lab, keepdims=True)
        vals = jnp.where(iota_l == idx, -jnp.inf, vals)
        return vals, a_s, a_x1, a_y1, a_x2, a_y2, a_l

    st = (vals, zero_o, zero_o, zero_o, zero_o, zero_o, zero_o)
    st = jax.lax.fori_loop(0, _DETS, sel_body, st)
    _, a_s, a_x1, a_y1, a_x2, a_y2, a_l = st
    osc_ref[...] = a_s
    ox1_ref[...] = a_x1
    oy1_ref[...] = a_y1
    ox2_ref[...] = a_x2
    oy2_ref[...] = a_y2
    olab_ref[...] = a_l


def _select_kernel(score_ref, t_ref, c_ref):
    s = score_ref[...]
    lane = jax.lax.broadcasted_iota(jnp.int32, s.shape, 1)
    row = jax.lax.broadcasted_iota(jnp.int32, s.shape, 0)
    lane_ok = (lane >= 1) & (lane < _NC)
    kbits = jax.lax.bitcast_convert_type(s, jnp.int32)
    int_min = jnp.int32(-2147483648)
    keys = jnp.where(lane_ok, kbits, int_min)
    fidx = jnp.where(lane_ok, row * (_NC - 1) + (lane - 1),
                     jnp.int32(2 ** 30))

    def bs1(_, st):
        lo, hi = st
        mid = (lo >> 1) + (hi >> 1) + (lo & hi & 1)
        cnt = jnp.sum((keys > mid).astype(jnp.int32))
        below = cnt < _TOPK
        return jnp.where(below, lo, mid + 1), jnp.where(below, mid, hi)

    lo, hi = jax.lax.fori_loop(
        0, 32, bs1, (int_min, jnp.int32(2147483647)))
    t = lo
    c_gt = jnp.sum((keys > t).astype(jnp.int32))
    m = _TOPK - c_gt
    tie = keys == t

    def bs2(_, st):
        lo, hi = st
        mid = (lo + hi) >> 1
        cnt = jnp.sum((tie & (fidx <= mid)).astype(jnp.int32))
        enough = cnt >= m
        return jnp.where(enough, lo, mid + 1), jnp.where(enough, mid, hi)

    lo2, _ = jax.lax.fori_loop(
        0, 19, bs2, (jnp.int32(0), jnp.int32(_N * (_NC - 1) - 1)))
    t_ref[...] = jnp.reshape(t, (1, 1))
    c_ref[...] = jnp.reshape(lo2, (1, 1))


def kernel(class_logits, box_regression, proposals):
    f32 = jnp.float32
    pad_c = _LANES - _NC
    logits_p = jnp.pad(class_logits, ((0, 0), (0, pad_c)),
                       constant_values=-1e30)
    rel = box_regression.reshape(_N, _NC, 4)
    dx = jnp.pad(rel[..., 0], ((0, 0), (0, pad_c)))
    dy = jnp.pad(rel[..., 1], ((0, 0), (0, pad_c)))
    dw = jnp.pad(rel[..., 2], ((0, 0), (0, pad_c)))
    dh = jnp.pad(rel[..., 3], ((0, 0), (0, pad_c)))
    px1 = proposals[:, 0:1]
    py1 = proposals[:, 1:2]
    px2 = proposals[:, 2:3]
    py2 = proposals[:, 3:4]

    n_blk = _N // _ROWS_BLK
    row_spec = pl.BlockSpec((_ROWS_BLK, _LANES), lambda i: (i, 0))
    col1_spec = pl.BlockSpec((_ROWS_BLK, 1), lambda i: (i, 0))
    big = jax.ShapeDtypeStruct((_N, _LANES), f32)
    score, bx1, by1, bx2, by2 = pl.pallas_call(
        _decode_kernel,
        grid=(n_blk,),
        in_specs=[row_spec] * 5 + [col1_spec] * 4,
        out_specs=[row_spec] * 5,
        out_shape=[big] * 5,
    )(logits_p, dx, dy, dw, dh, px1, py1, px2, py2)

    scores_flat = score[:, 1:_NC].reshape(-1)
    x1f = bx1[:, 1:_NC].reshape(-1)
    y1f = by1[:, 1:_NC].reshape(-1)
    x2f = bx2[:, 1:_NC].reshape(-1)
    y2f = by2[:, 1:_NC].reshape(-1)

    t_arr, c_arr = pl.pallas_call(
        _select_kernel,
        out_shape=[jax.ShapeDtypeStruct((1, 1), jnp.int32)] * 2,
    )(score)
    t_key = t_arr[0, 0]
    c_cut = c_arr[0, 0]
    keys_flat = jax.lax.bitcast_convert_type(scores_flat, jnp.int32)
    pred = (keys_flat > t_key) | (
        (keys_flat == t_key)
        & (jnp.arange(keys_flat.shape[0], dtype=jnp.int32) <= c_cut))
    cidx = jnp.nonzero(pred, size=_TOPK, fill_value=0)[0].astype(jnp.int32)
    top_scores, pos = jax.lax.top_k(scores_flat[cidx], _TOPK)
    top_idx = cidx[pos]
    tlab = (top_idx % (_NC - 1) + 1).astype(f32)
    tx1 = x1f[top_idx]
    ty1 = y1f[top_idx]
    tx2 = x2f[top_idx]
    ty2 = y2f[top_idx]

    npad = _PAD - _TOPK
    def cpad(v, cval=0.0):
        return jnp.pad(v, (0, npad), constant_values=cval).reshape(1, _PAD)
    sco_c = cpad(top_scores, -1.0)
    x1c, y1c, x2c, y2c = cpad(tx1), cpad(ty1), cpad(tx2), cpad(ty2)
    lab_c = cpad(tlab)
    x1r, y1r = x1c.reshape(_PAD, 1), y1c.reshape(_PAD, 1)
    x2r, y2r = x2c.reshape(_PAD, 1), y2c.reshape(_PAD, 1)
    lab_r = lab_c.reshape(_PAD, 1)

    rblk = pl.BlockSpec((_S_BLK, 1), lambda i: (i, 0))
    cblk = pl.BlockSpec((1, _PAD), lambda i: (0, 0))
    s_mat = pl.pallas_call(
        _smat_kernel,
        grid=(_PAD // _S_BLK,),
        in_specs=[rblk] * 5 + [cblk] * 5,
        out_specs=pl.BlockSpec((_S_BLK, _PAD), lambda i: (i, 0)),
        out_shape=jax.ShapeDtypeStruct((_PAD, _PAD), f32),
    )(x1r, y1r, x2r, y2r, lab_r, x1c, y1c, x2c, y2c, lab_c)

    out_small = jax.ShapeDtypeStruct((1, _LANES), f32)
    osc, ox1, oy1, ox2, oy2, olab = pl.pallas_call(
        _nms_kernel,
        out_shape=[out_small] * 6,
    )(s_mat, sco_c, x1c, y1c, x2c, y2c, lab_c)

    final_scores = osc[0, :_DETS]
    final_boxes = jnp.stack(
        [ox1[0, :_DETS], oy1[0, :_DETS], ox2[0, :_DETS], oy2[0, :_DETS]],
        axis=-1)
    final_labels = olab[0, :_DETS].astype(jnp.int32)
    return final_boxes, final_scores, final_labels
